# baseline (speedup 1.0000x reference)
import functools

import jax
import jax.numpy as jnp
from jax import lax
from jax.experimental import pallas as pl
from jax.experimental.pallas import tpu as pltpu
from jax.experimental.pallas import tpu_sc as plsc

SEQ = 200
BATCH = 4096
DIM = 64
VOCAB = 1000000
B = SEQ * BATCH
NC = 2
NS = 16
NW = NC * NS
SUB = 128
B_PER_W = B // NW
N_CHUNKS = B_PER_W // SUB
NBUF = 4
NTB = 2
BTC = BATCH // SUB

_mesh = plsc.VectorSubcoreMesh(core_axis_name="c", subcore_axis_name="s")


@functools.partial(
    pl.kernel,
    mesh=_mesh,
    out_type=jax.ShapeDtypeStruct((SEQ, 8, BTC, 8, SUB), jnp.float32),
    compiler_params=pltpu.CompilerParams(
        use_tc_tiling_on_sc=False, needs_layout_passes=False),
    name="sc_embedding_gather",
    scratch_types=[
        pltpu.VMEM((N_CHUNKS, SUB), jnp.int32),
        [pltpu.VMEM((SUB, DIM), jnp.float32)] * NBUF,
        [pltpu.VMEM((8, 8, SUB), jnp.float32)] * NTB,
        [pltpu.SemaphoreType.DMA] * NBUF,
        [pltpu.SemaphoreType.DMA] * NTB,
    ],
)
def _gather_kernel(idx_hbm, table_hbm, out_hbm, idx_v, bufs, tbufs,
                   gsems, wsems):
    wid = lax.axis_index("s") * NC + lax.axis_index("c")

    idx_base = pl.multiple_of(wid * N_CHUNKS, 8)
    pltpu.sync_copy(idx_hbm.at[pl.ds(idx_base, N_CHUNKS)], idx_v)

    def fire_gather(c, b):
        pltpu.async_copy(table_hbm.at[idx_v.at[c]], bufs[b], gsems[b])

    def wait_gather(b):
        pltpu.make_async_copy(
            table_hbm.at[pl.ds(0, SUB)], bufs[b], gsems[b]).wait()

    def fire_tile_write(c, tb):
        g = wid * N_CHUNKS + c
        s = g // BTC
        btc = g % BTC
        pltpu.async_copy(tbufs[tb], out_hbm.at[s, :, btc], wsems[tb])

    def wait_tile_write(tb):
        pltpu.make_async_copy(
            tbufs[tb], out_hbm.at[0, :, 0], wsems[tb]).wait()

    iota16 = lax.iota(jnp.int32, 16)

    def transpose_chunk(b, tb):
        @plsc.parallel_loop(0, 8, step=1, unroll=2)
        def _qbody(q):
            rows = iota16 + q * 16
            base = q * 16
            for dtr in range(8):
                for dr in range(8):
                    cols = jnp.full((16,), dtr * 8 + dr, jnp.int32)
                    v = plsc.load_gather(bufs[b], [rows, cols])
                    tbufs[tb][dtr, dr, pl.ds(base, 16)] = v

    for b in range(NBUF - 1):
        fire_gather(b, b)

    def body(k, carry):
        for j in range(NBUF):
            c = k * NBUF + j
            tb = j % NTB
            wait_gather(j)
            if j < NTB:
                @pl.when(k > 0)
                def _():
                    wait_tile_write(tb)
            else:
                wait_tile_write(tb)
            transpose_chunk(j, tb)
            nb = (j + NBUF - 1) % NBUF
            if j == 0:
                fire_gather(c + NBUF - 1, nb)
            else:
                @pl.when(k < N_CHUNKS // NBUF - 1)
                def _():
                    fire_gather(c + NBUF - 1, nb)
            fire_tile_write(c, tb)
        return carry

    lax.fori_loop(0, N_CHUNKS // NBUF, body, 0)
    wait_tile_write(0)
    wait_tile_write(1)


def kernel(source, table):
    idx2 = source.reshape(B // SUB, SUB) * 2
    tpad = jnp.pad(table, ((0, 0), (0, DIM))).reshape(2 * VOCAB, DIM)
    out5 = _gather_kernel(idx2, tpad)
    return out5.transpose(0, 2, 4, 1, 3).reshape(SEQ, BATCH, 1, DIM)


# speedup vs baseline: 1.2025x; 1.2025x over previous
import functools

import jax
import jax.numpy as jnp
from jax import lax
from jax.experimental import pallas as pl
from jax.experimental.pallas import tpu as pltpu
from jax.experimental.pallas import tpu_sc as plsc

SEQ = 200
BATCH = 4096
DIM = 64
VOCAB = 1000000
B = SEQ * BATCH
NC = 2
NS = 16
NW = NC * NS
B_PER_W = B // NW
SUB = 128
N_SUB = 2
CHUNK = SUB * N_SUB
NBUF = 4
H = B_PER_W // CHUNK
K = H // NBUF
IDX_ROWS = B_PER_W // SUB

_mesh = plsc.VectorSubcoreMesh(core_axis_name="c", subcore_axis_name="s")


@functools.partial(
    pl.kernel,
    mesh=_mesh,
    out_type=jax.ShapeDtypeStruct((B, DIM), jnp.float32),
    compiler_params=pltpu.CompilerParams(use_tc_tiling_on_sc=False),
    name="sc_embedding_gather",
    scratch_types=[
        pltpu.VMEM((IDX_ROWS, SUB), jnp.int32),
        [pltpu.VMEM((CHUNK, DIM), jnp.float32)] * NBUF,
        [pltpu.SemaphoreType.DMA] * NBUF,
        [pltpu.SemaphoreType.DMA] * NBUF,
    ],
)
def _gather_kernel(idx_hbm, table_hbm, out_hbm, idx_v, bufs, gsems, wsems):
    wid = lax.axis_index("s") * NC + lax.axis_index("c")
    base = wid * B_PER_W

    idx_base = pl.multiple_of(wid * IDX_ROWS, 8)
    pltpu.sync_copy(idx_hbm.at[pl.ds(idx_base, IDX_ROWS)], idx_v)

    def fire_gathers(h, b):
        for j in range(N_SUB):
            pltpu.async_copy(
                table_hbm.at[idx_v.at[h * N_SUB + j]],
                bufs[b].at[pl.ds(j * SUB, SUB)],
                gsems[b],
            )

    def wait_gathers(b):
        pltpu.make_async_copy(
            table_hbm.at[pl.ds(0, CHUNK)], bufs[b], gsems[b]).wait()

    def fire_write(h, b):
        off = pl.multiple_of(base + h * CHUNK, 8)
        pltpu.async_copy(bufs[b], out_hbm.at[pl.ds(off, CHUNK)], wsems[b])

    def wait_write(b):
        pltpu.make_async_copy(
            bufs[b], out_hbm.at[pl.ds(0, CHUNK)], wsems[b]).wait()

    for b in range(NBUF - 1):
        fire_gathers(b, b)

    def body(k, carry):
        for j in range(NBUF):
            h = k * NBUF + j
            nb = (j + NBUF - 1) % NBUF
            wait_gathers(j)
            if j == 0:
                @pl.when(k > 0)
                def _():
                    wait_write(nb)
            else:
                wait_write(nb)
            if j == 0:
                fire_gathers(h + NBUF - 1, nb)
            else:
                @pl.when(k < K - 1)
                def _():
                    fire_gathers(h + NBUF - 1, nb)
            fire_write(h, j)
        return carry

    lax.fori_loop(0, K, body, 0)
    wait_write(NBUF - 1)


def kernel(source, table):
    idx2 = source.reshape(B // SUB, SUB) * 2
    tpad = jnp.pad(table, ((0, 0), (0, DIM))).reshape(2 * VOCAB, DIM)
    out = _gather_kernel(idx2, tpad)
    return out.reshape(SEQ, BATCH, 1, DIM)
